# baseline (speedup 1.0000x reference)
import functools

import jax
import jax.numpy as jnp
from jax import lax
from jax.experimental import pallas as pl
from jax.experimental.pallas import tpu as pltpu
from jax.experimental.pallas import tpu_sc as plsc

N = 10000
E = 320000
DF = 128
DH = 16

NC = 2
NS = 16
NW = NC * NS
ECB = 512
NTIL = ECB // 128
TOTCH = E // ECB
CPT = (TOTCH + NW - 1) // NW
RPT = N // NS
ET = E // 128


def _xs_body(x_ref, w_ref, o_ref):
    o_ref[...] = jnp.dot(x_ref[...], w_ref[...].T,
                         preferred_element_type=jnp.float32)


def _node_pre(x, mw1a):
    bm = 2000
    return pl.pallas_call(
        _xs_body,
        grid=(N // bm,),
        in_specs=[pl.BlockSpec((bm, DF), lambda i: (i, 0)),
                  pl.BlockSpec((DH, DF), lambda i: (0, 0))],
        out_specs=pl.BlockSpec((bm, DH), lambda i: (i, 0)),
        out_shape=jax.ShapeDtypeStruct((N, DH), jnp.float32),
    )(x, mw1a)


def _eat_body(ea_ref, w_ref, b_ref, o_ref):
    for k in range(o_ref.shape[0]):
        blk = jnp.dot(w_ref[...], ea_ref[:, k * 128:(k + 1) * 128],
                      preferred_element_type=jnp.float32) + b_ref[...]
        o_ref[k, :, :] = blk


def _eat_tc(ea_t, w1b, b1):
    bm = 12800
    kt = bm // 128
    return pl.pallas_call(
        _eat_body,
        grid=(E // bm,),
        in_specs=[pl.BlockSpec((DH, bm), lambda i: (0, i)),
                  pl.BlockSpec((DH, DH), lambda i: (0, 0)),
                  pl.BlockSpec((DH, 1), lambda i: (0, 0))],
        out_specs=pl.BlockSpec((kt, DH, 128), lambda i: (i, 0, 0)),
        out_shape=jax.ShapeDtypeStruct((ET, DH, 128), jnp.float32),
    )(ea_t, w1b, b1)


def _node_body(x_ref, p0_ref, p1_ref, w1a_ref, w1b_ref, b1_ref, w2_ref,
               b2_ref, w3_ref, b3_ref, o_ref):
    agg = p0_ref[...] + p1_ref[...]
    t = (jnp.dot(x_ref[...], w1a_ref[...].T, preferred_element_type=jnp.float32)
         + jnp.dot(agg, w1b_ref[...].T, preferred_element_type=jnp.float32)
         + b1_ref[...])
    t = jnp.maximum(t, 0.0)
    t = jnp.dot(t, w2_ref[...].T, preferred_element_type=jnp.float32) + b2_ref[...]
    t = jnp.maximum(t, 0.0)
    o_ref[...] = jnp.dot(t, w3_ref[...].T,
                         preferred_element_type=jnp.float32) + b3_ref[...]


def _node_mlp(x, parts, w1a, w1b, b1, w2, b2, w3, b3):
    bm = 2000
    nb = N // bm
    wspec = pl.BlockSpec((DH, DH), lambda i: (0, 0))
    bspec = pl.BlockSpec((1, DH), lambda i: (0, 0))
    return pl.pallas_call(
        _node_body,
        grid=(nb,),
        in_specs=[pl.BlockSpec((bm, DF), lambda i: (i, 0)),
                  pl.BlockSpec((bm, DH), lambda i: (i, 0)),
                  pl.BlockSpec((bm, DH), lambda i: (i + nb, 0)),
                  pl.BlockSpec((DH, DF), lambda i: (0, 0)),
                  wspec, bspec, wspec, bspec, wspec, bspec],
        out_specs=pl.BlockSpec((bm, DH), lambda i: (i, 0)),
        out_shape=jax.ShapeDtypeStruct((N, DH), jnp.float32),
    )(x, parts, parts, w1a, w1b, b1, w2, b2, w3, b3)


def _edge_sc(xs, eat, send, rec, w2, b2, w3, b3):
    mesh = plsc.VectorSubcoreMesh(core_axis_name="c", subcore_axis_name="s")

    @functools.partial(
        pl.kernel,
        out_type=jax.ShapeDtypeStruct((NC * N, DH), jnp.float32),
        mesh=mesh,
        scratch_types=[pltpu.VMEM_SHARED((N, DH), jnp.float32),
                       pltpu.VMEM((RPT, DH), jnp.float32),
                       pltpu.VMEM((CPT * ECB,), jnp.int32),
                       pltpu.VMEM((CPT, ECB), jnp.int32),
                       pltpu.VMEM((ECB, DH), jnp.float32),
                       pltpu.VMEM((NTIL, DH, 128), jnp.float32),
                       pltpu.VMEM((ECB, DH), jnp.float32),
                       pltpu.VMEM((DH * DH, 16), jnp.float32),
                       pltpu.VMEM((DH, 16), jnp.float32),
                       pltpu.VMEM((DH * DH, 16), jnp.float32),
                       pltpu.VMEM((DH, 16), jnp.float32),
                       pltpu.VMEM((DH, 2, 16), jnp.float32),
                       pltpu.VMEM((DH, 2, 16), jnp.float32),
                       pltpu.SemaphoreType.DMA,
                       pltpu.SemaphoreType.DMA,
                       pltpu.SemaphoreType.DMA,
                       pltpu.SemaphoreType.DMA],
        compiler_params=pltpu.CompilerParams(use_tc_tiling_on_sc=False,
                                             needs_layout_passes=False),
    )
    def k(xs_hbm, eat_hbm, send_hbm, rec_hbm, w2_hbm, b2_hbm, w3_hbm, b3_hbm,
          out_hbm, agg_sh, zrows_v, sidx_v, recb_v, xg_v, eat_v, m3_v,
          w2_v, b2_v, w3_v, b3_v, m1_v, m2_v, isem, rsem, gsem, esem):
        cid = lax.axis_index("c")
        sid = lax.axis_index("s")
        wid = sid * NC + cid

        pltpu.sync_copy(w2_hbm, w2_v)
        pltpu.sync_copy(b2_hbm, b2_v)
        pltpu.sync_copy(w3_hbm, w3_v)
        pltpu.sync_copy(b3_hbm, b3_v)

        for j in range(CPT):
            c = wid + NW * j
            off = jnp.where(c < TOTCH, c * ECB, 0)
            pltpu.async_copy(send_hbm.at[pl.ds(off, ECB)],
                             sidx_v.at[pl.ds(j * ECB, ECB)], isem)
            pltpu.async_copy(rec_hbm.at[pl.ds(off, ECB)],
                             recb_v.at[j], rsem)
        pltpu.make_async_copy(send_hbm.at[pl.ds(0, CPT * ECB)],
                              sidx_v, isem).wait()
        for j in range(CPT):
            pltpu.make_async_copy(rec_hbm.at[pl.ds(0, ECB)],
                                  recb_v.at[j], rsem).wait()

        def zero_body(r, carry):
            zrows_v[r, :] = jnp.zeros((DH,), jnp.float32)
            return carry

        lax.fori_loop(0, RPT, zero_body, 0)
        pltpu.sync_copy(zrows_v, agg_sh.at[pl.ds(sid * RPT, RPT)])
        plsc.subcore_barrier()

        iota = lax.iota(jnp.int32, 16)

        def do_chunk(j, carry):
            c = wid + NW * j

            @pl.when(c < TOTCH)
            def _():
                pltpu.async_copy(
                    xs_hbm.at[sidx_v.at[pl.ds(j * ECB, ECB)]], xg_v,
                    gsem).wait()
                pltpu.async_copy(eat_hbm.at[pl.ds(NTIL * c, NTIL)],
                                 eat_v, esem).wait()

                def pair(t, carry2):
                    eidx0 = (2 * t) * 16 + iota
                    eidx1 = (2 * t + 1) * 16 + iota
                    for gg, eidx in ((0, eidx0), (1, eidx1)):
                        g = 2 * t + gg
                        ktile = g // 8
                        coff = (g % 8) * 16
                        for f in range(DH):
                            xgf = plsc.load_gather(
                                xg_v, [eidx, jnp.full((16,), f, jnp.int32)])
                            ef = eat_v[ktile, f, pl.ds(coff, 16)]
                            m1_v[f, gg, :] = jnp.maximum(xgf + ef, 0.0)
                    acc = [[b2_v[o, :], b2_v[o, :]] for o in range(DH)]
                    for f in range(DH):
                        a0 = m1_v[f, 0, :]
                        a1 = m1_v[f, 1, :]
                        for o in range(DH):
                            w = w2_v[o * DH + f, :]
                            acc[o][0] = acc[o][0] + a0 * w
                            acc[o][1] = acc[o][1] + a1 * w
                    for o in range(DH):
                        m2_v[o, 0, :] = jnp.maximum(acc[o][0], 0.0)
                        m2_v[o, 1, :] = jnp.maximum(acc[o][1], 0.0)
                    acc = [[b3_v[o, :], b3_v[o, :]] for o in range(DH)]
                    for f in range(DH):
                        a0 = m2_v[f, 0, :]
                        a1 = m2_v[f, 1, :]
                        for o in range(DH):
                            w = w3_v[o * DH + f, :]
                            acc[o][0] = acc[o][0] + a0 * w
                            acc[o][1] = acc[o][1] + a1 * w
                    for o in range(DH):
                        ov = jnp.full((16,), o, jnp.int32)
                        plsc.store_scatter(m3_v, [eidx0, ov], acc[o][0])
                        plsc.store_scatter(m3_v, [eidx1, ov], acc[o][1])
                    return carry2

                lax.fori_loop(0, ECB // 32, pair, 0)
                pltpu.sync_copy(m3_v, agg_sh.at[recb_v.at[j]], add=True)

            return carry

        lax.fori_loop(0, CPT, do_chunk, 0)
        plsc.subcore_barrier()
        pltpu.sync_copy(agg_sh.at[pl.ds(sid * RPT, RPT)],
                        out_hbm.at[pl.ds(cid * N + sid * RPT, RPT)])

    return k(xs, eat, send, rec, w2, b2, w3, b3)


def kernel(x, edge_index, edge_attr, u, batch, mw1, mb1, mw2, mb2, mw3, mb3,
           nw1, nb1, nw2, nb2, nw3, nb3):
    send = edge_index[0]
    rec = edge_index[1]
    mw1a = mw1[:, :DF]
    mw1b = mw1[:, DF:]
    nw1a = nw1[:, :DF]
    nw1b = nw1[:, DF:]

    ones16 = jnp.ones((1, 16), jnp.float32)
    wb2 = mw2.reshape(DH * DH, 1) * ones16
    wb3 = mw3.reshape(DH * DH, 1) * ones16
    bb2 = mb2.reshape(DH, 1) * ones16
    bb3 = mb3.reshape(DH, 1) * ones16

    xs = _node_pre(x, mw1a)
    eat = _eat_tc(edge_attr.T, mw1b, mb1.reshape(DH, 1))
    parts = _edge_sc(xs, eat, send, rec, wb2, bb2, wb3, bb3)
    h = _node_mlp(x, parts, nw1a, nw1b, nb1.reshape(1, DH),
                  nw2, nb2.reshape(1, DH), nw3, nb3.reshape(1, DH))
    return h


# speedup vs baseline: 1.8260x; 1.8260x over previous
import functools

import jax
import jax.numpy as jnp
from jax import lax
from jax.experimental import pallas as pl
from jax.experimental.pallas import tpu as pltpu
from jax.experimental.pallas import tpu_sc as plsc

N = 10000
E = 320000
DF = 128
DH = 16

NC = 2
NS = 16
NW = NC * NS
ET = E // 128
CET = 20
ECB = CET * 128
TOTCH = ET // CET
CPT = (TOTCH + NW - 1) // NW
RPT = N // NS


def _xs_body(x_ref, w_ref, o_ref):
    o_ref[...] = jnp.dot(x_ref[...], w_ref[...].T,
                         preferred_element_type=jnp.float32)


def _node_pre(x, mw1a):
    bm = 2000
    return pl.pallas_call(
        _xs_body,
        grid=(N // bm,),
        in_specs=[pl.BlockSpec((bm, DF), lambda i: (i, 0)),
                  pl.BlockSpec((DH, DF), lambda i: (0, 0))],
        out_specs=pl.BlockSpec((bm, DH), lambda i: (i, 0)),
        out_shape=jax.ShapeDtypeStruct((N, DH), jnp.float32),
    )(x, mw1a)


def _edge_body(ea_ref, xg_ref, w1_ref, b1_ref, w2_ref, b2_ref, w3_ref,
               b3_ref, o_ref):
    for k in range(o_ref.shape[0]):
        eat = jnp.dot(w1_ref[...], ea_ref[:, k * 128:(k + 1) * 128],
                      preferred_element_type=jnp.float32) + b1_ref[...]
        t = jnp.maximum(xg_ref[k, :, :] + eat, 0.0)
        t = jnp.maximum(
            jnp.dot(w2_ref[...], t, preferred_element_type=jnp.float32)
            + b2_ref[...], 0.0)
        o_ref[k, :, :] = jnp.dot(
            w3_ref[...], t, preferred_element_type=jnp.float32) + b3_ref[...]


def _edge_tc(ea_t, xgt, w1b, b1, w2, b2, w3, b3):
    kt = CET
    bm = kt * 128
    wspec = pl.BlockSpec((DH, DH), lambda i: (0, 0))
    bspec = pl.BlockSpec((DH, 1), lambda i: (0, 0))
    return pl.pallas_call(
        _edge_body,
        grid=(ET // kt,),
        in_specs=[pl.BlockSpec((DH, bm), lambda i: (0, i)),
                  pl.BlockSpec((kt, DH, 128), lambda i: (i, 0, 0)),
                  wspec, bspec, wspec, bspec, wspec, bspec],
        out_specs=pl.BlockSpec((kt, DH, 128), lambda i: (i, 0, 0)),
        out_shape=jax.ShapeDtypeStruct((ET, DH, 128), jnp.float32),
    )(ea_t, xgt, w1b, b1, w2, b2, w3, b3)


def _node_body(x_ref, p0_ref, p1_ref, w1a_ref, w1b_ref, b1_ref, w2_ref,
               b2_ref, w3_ref, b3_ref, o_ref):
    agg = p0_ref[...] + p1_ref[...]
    t = (jnp.dot(x_ref[...], w1a_ref[...].T, preferred_element_type=jnp.float32)
         + jnp.dot(agg, w1b_ref[...].T, preferred_element_type=jnp.float32)
         + b1_ref[...])
    t = jnp.maximum(t, 0.0)
    t = jnp.dot(t, w2_ref[...].T, preferred_element_type=jnp.float32) + b2_ref[...]
    t = jnp.maximum(t, 0.0)
    o_ref[...] = jnp.dot(t, w3_ref[...].T,
                         preferred_element_type=jnp.float32) + b3_ref[...]


def _node_mlp(x, parts, w1a, w1b, b1, w2, b2, w3, b3):
    bm = 2000
    nb = N // bm
    wspec = pl.BlockSpec((DH, DH), lambda i: (0, 0))
    bspec = pl.BlockSpec((1, DH), lambda i: (0, 0))
    return pl.pallas_call(
        _node_body,
        grid=(nb,),
        in_specs=[pl.BlockSpec((bm, DF), lambda i: (i, 0)),
                  pl.BlockSpec((bm, DH), lambda i: (i, 0)),
                  pl.BlockSpec((bm, DH), lambda i: (i + nb, 0)),
                  pl.BlockSpec((DH, DF), lambda i: (0, 0)),
                  wspec, bspec, wspec, bspec, wspec, bspec],
        out_specs=pl.BlockSpec((bm, DH), lambda i: (i, 0)),
        out_shape=jax.ShapeDtypeStruct((N, DH), jnp.float32),
    )(x, parts, parts, w1a, w1b, b1, w2, b2, w3, b3)


def _gather_sc(xs, send):
    mesh = plsc.VectorSubcoreMesh(core_axis_name="c", subcore_axis_name="s")

    @functools.partial(
        pl.kernel,
        out_type=jax.ShapeDtypeStruct((ET, DH, 128), jnp.float32),
        mesh=mesh,
        scratch_types=[pltpu.VMEM((CPT * ECB,), jnp.int32),
                       pltpu.VMEM((ECB, DH), jnp.float32),
                       pltpu.VMEM((CET, DH, 128), jnp.float32),
                       pltpu.SemaphoreType.DMA,
                       pltpu.SemaphoreType.DMA],
        compiler_params=pltpu.CompilerParams(use_tc_tiling_on_sc=False,
                                             needs_layout_passes=False),
    )
    def k(xs_hbm, send_hbm, out_hbm, sidx_v, xg_v, xgt_v, isem, gsem):
        wid = lax.axis_index("s") * NC + lax.axis_index("c")

        for j in range(CPT):
            c = wid + NW * j
            off = jnp.where(c < TOTCH, c * ECB, 0)
            pltpu.async_copy(send_hbm.at[pl.ds(off, ECB)],
                             sidx_v.at[pl.ds(j * ECB, ECB)], isem)
        pltpu.make_async_copy(send_hbm.at[pl.ds(0, CPT * ECB)],
                              sidx_v, isem).wait()

        iota = lax.iota(jnp.int32, 16)

        def do_chunk(j, carry):
            c = wid + NW * j

            @pl.when(c < TOTCH)
            def _():
                pltpu.async_copy(
                    xs_hbm.at[sidx_v.at[pl.ds(j * ECB, ECB)]], xg_v,
                    gsem).wait()

                def group(g, carry2):
                    eidx = g * 16 + iota
                    ktile = g // 8
                    coff = (g % 8) * 16
                    for f in range(DH):
                        v = plsc.load_gather(
                            xg_v, [eidx, jnp.full((16,), f, jnp.int32)])
                        xgt_v[ktile, f, pl.ds(coff, 16)] = v
                    return carry2

                lax.fori_loop(0, ECB // 16, group, 0)
                pltpu.sync_copy(xgt_v, out_hbm.at[pl.ds(CET * c, CET)])

            return carry

        lax.fori_loop(0, CPT, do_chunk, 0)

    return k(xs, send)


def _scatter_sc(m3t, rec):
    mesh = plsc.VectorSubcoreMesh(core_axis_name="c", subcore_axis_name="s")

    @functools.partial(
        pl.kernel,
        out_type=jax.ShapeDtypeStruct((NC * N, DH), jnp.float32),
        mesh=mesh,
        scratch_types=[pltpu.VMEM_SHARED((N, DH), jnp.float32),
                       pltpu.VMEM((RPT, DH), jnp.float32),
                       pltpu.VMEM((CPT, ECB), jnp.int32),
                       pltpu.VMEM((CET, DH, 128), jnp.float32),
                       pltpu.VMEM((ECB, DH), jnp.float32),
                       pltpu.SemaphoreType.DMA,
                       pltpu.SemaphoreType.DMA],
        compiler_params=pltpu.CompilerParams(use_tc_tiling_on_sc=False,
                                             needs_layout_passes=False),
    )
    def k(m3_hbm, rec_hbm, out_hbm, agg_sh, zrows_v, recb_v, m3t_v, m3_v,
          rsem, lsem):
        cid = lax.axis_index("c")
        sid = lax.axis_index("s")
        wid = sid * NC + cid

        for j in range(CPT):
            c = wid + NW * j
            off = jnp.where(c < TOTCH, c * ECB, 0)
            pltpu.async_copy(rec_hbm.at[pl.ds(off, ECB)], recb_v.at[j], rsem)
        for j in range(CPT):
            pltpu.make_async_copy(rec_hbm.at[pl.ds(0, ECB)],
                                  recb_v.at[j], rsem).wait()

        def zero_body(r, carry):
            zrows_v[r, :] = jnp.zeros((DH,), jnp.float32)
            return carry

        lax.fori_loop(0, RPT, zero_body, 0)
        pltpu.sync_copy(zrows_v, agg_sh.at[pl.ds(sid * RPT, RPT)])
        plsc.subcore_barrier()

        iota = lax.iota(jnp.int32, 16)

        def do_chunk(j, carry):
            c = wid + NW * j

            @pl.when(c < TOTCH)
            def _():
                pltpu.async_copy(m3_hbm.at[pl.ds(CET * c, CET)], m3t_v,
                                 lsem).wait()

                def group(g, carry2):
                    eidx = g * 16 + iota
                    ktile = g // 8
                    coff = (g % 8) * 16
                    for f in range(DH):
                        v = m3t_v[ktile, f, pl.ds(coff, 16)]
                        plsc.store_scatter(
                            m3_v, [eidx, jnp.full((16,), f, jnp.int32)], v)
                    return carry2

                lax.fori_loop(0, ECB // 16, group, 0)
                pltpu.sync_copy(m3_v, agg_sh.at[recb_v.at[j]], add=True)

            return carry

        lax.fori_loop(0, CPT, do_chunk, 0)
        plsc.subcore_barrier()
        pltpu.sync_copy(agg_sh.at[pl.ds(sid * RPT, RPT)],
                        out_hbm.at[pl.ds(cid * N + sid * RPT, RPT)])

    return k(m3t, rec)


def kernel(x, edge_index, edge_attr, u, batch, mw1, mb1, mw2, mb2, mw3, mb3,
           nw1, nb1, nw2, nb2, nw3, nb3):
    send = edge_index[0]
    rec = edge_index[1]
    mw1a = mw1[:, :DF]
    mw1b = mw1[:, DF:]
    nw1a = nw1[:, :DF]
    nw1b = nw1[:, DF:]

    xs = _node_pre(x, mw1a)
    xgt = _gather_sc(xs, send)
    m3t = _edge_tc(edge_attr.T, xgt, mw1b, mb1.reshape(DH, 1),
                   mw2, mb2.reshape(DH, 1), mw3, mb3.reshape(DH, 1))
    parts = _scatter_sc(m3t, rec)
    h = _node_mlp(x, parts, nw1a, nw1b, nb1.reshape(1, DH),
                  nw2, nb2.reshape(1, DH), nw3, nb3.reshape(1, DH))
    return h


# speedup vs baseline: 3.1021x; 1.6989x over previous
import functools

import jax
import jax.numpy as jnp
from jax import lax
from jax.experimental import pallas as pl
from jax.experimental.pallas import tpu as pltpu
from jax.experimental.pallas import tpu_sc as plsc

N = 10000
E = 320000
DF = 128
DH = 16

NC = 2
NS = 16
NW = NC * NS
ET = E // 128
CET = 20
ECB = CET * 128
TOTCH = ET // CET
CPT = (TOTCH + NW - 1) // NW
RPT = N // NS


def _xs_body(x_ref, w_ref, o_ref):
    o_ref[...] = jnp.dot(x_ref[...], w_ref[...].T,
                         preferred_element_type=jnp.float32)


def _node_pre(x, mw1a):
    bm = 2000
    return pl.pallas_call(
        _xs_body,
        grid=(N // bm,),
        in_specs=[pl.BlockSpec((bm, DF), lambda i: (i, 0)),
                  pl.BlockSpec((DH, DF), lambda i: (0, 0))],
        out_specs=pl.BlockSpec((bm, DH), lambda i: (i, 0)),
        out_shape=jax.ShapeDtypeStruct((N, DH), jnp.float32),
    )(x, mw1a)


def _eat_body(ea_ref, w_ref, b_ref, o_ref):
    for k in range(o_ref.shape[0]):
        blk = jnp.dot(w_ref[...], ea_ref[:, k * 128:(k + 1) * 128],
                      preferred_element_type=jnp.float32) + b_ref[...]
        o_ref[k, :, :] = blk


def _eat_tc(ea_t, w1b, b1):
    bm = 12800
    kt = bm // 128
    return pl.pallas_call(
        _eat_body,
        grid=(E // bm,),
        in_specs=[pl.BlockSpec((DH, bm), lambda i: (0, i)),
                  pl.BlockSpec((DH, DH), lambda i: (0, 0)),
                  pl.BlockSpec((DH, 1), lambda i: (0, 0))],
        out_specs=pl.BlockSpec((kt, DH, 128), lambda i: (i, 0, 0)),
        out_shape=jax.ShapeDtypeStruct((ET, DH, 128), jnp.float32),
    )(ea_t, w1b, b1)


KB = CET * DH


def _edge_body(eat_ref, xg_ref, w2_ref, b2_ref, w3_ref, b3_ref, o_ref):
    t = jnp.maximum(xg_ref[...] + eat_ref[...], 0.0).reshape(KB, 128)
    t = jnp.maximum(
        jnp.dot(w2_ref[...], t, preferred_element_type=jnp.float32)
        + b2_ref[...], 0.0)
    t = jnp.dot(w3_ref[...], t, preferred_element_type=jnp.float32) + b3_ref[...]
    o_ref[...] = t.reshape(CET, DH, 128)


def _edge_tc(eat, xgt, w2k, b2k, w3k, b3k):
    kt = CET
    wspec = pl.BlockSpec((KB, KB), lambda i: (0, 0))
    bspec = pl.BlockSpec((KB, 1), lambda i: (0, 0))
    return pl.pallas_call(
        _edge_body,
        grid=(ET // kt,),
        in_specs=[pl.BlockSpec((kt, DH, 128), lambda i: (i, 0, 0)),
                  pl.BlockSpec((kt, DH, 128), lambda i: (i, 0, 0)),
                  wspec, bspec, wspec, bspec],
        out_specs=pl.BlockSpec((kt, DH, 128), lambda i: (i, 0, 0)),
        out_shape=jax.ShapeDtypeStruct((ET, DH, 128), jnp.float32),
    )(eat, xgt, w2k, b2k, w3k, b3k)


def _node_body(x_ref, p0_ref, p1_ref, w1a_ref, w1b_ref, b1_ref, w2_ref,
               b2_ref, w3_ref, b3_ref, o_ref):
    agg = p0_ref[...] + p1_ref[...]
    t = (jnp.dot(x_ref[...], w1a_ref[...].T, preferred_element_type=jnp.float32)
         + jnp.dot(agg, w1b_ref[...].T, preferred_element_type=jnp.float32)
         + b1_ref[...])
    t = jnp.maximum(t, 0.0)
    t = jnp.dot(t, w2_ref[...].T, preferred_element_type=jnp.float32) + b2_ref[...]
    t = jnp.maximum(t, 0.0)
    o_ref[...] = jnp.dot(t, w3_ref[...].T,
                         preferred_element_type=jnp.float32) + b3_ref[...]


def _node_mlp(x, parts, w1a, w1b, b1, w2, b2, w3, b3):
    bm = 2000
    nb = N // bm
    wspec = pl.BlockSpec((DH, DH), lambda i: (0, 0))
    bspec = pl.BlockSpec((1, DH), lambda i: (0, 0))
    return pl.pallas_call(
        _node_body,
        grid=(nb,),
        in_specs=[pl.BlockSpec((bm, DF), lambda i: (i, 0)),
                  pl.BlockSpec((bm, DH), lambda i: (i, 0)),
                  pl.BlockSpec((bm, DH), lambda i: (i + nb, 0)),
                  pl.BlockSpec((DH, DF), lambda i: (0, 0)),
                  wspec, bspec, wspec, bspec, wspec, bspec],
        out_specs=pl.BlockSpec((bm, DH), lambda i: (i, 0)),
        out_shape=jax.ShapeDtypeStruct((N, DH), jnp.float32),
    )(x, parts, parts, w1a, w1b, b1, w2, b2, w3, b3)


def _gather_sc(xs, send):
    mesh = plsc.VectorSubcoreMesh(core_axis_name="c", subcore_axis_name="s")

    @functools.partial(
        pl.kernel,
        out_type=jax.ShapeDtypeStruct((ET, DH, 128), jnp.float32),
        mesh=mesh,
        scratch_types=[pltpu.VMEM((CPT * ECB,), jnp.int32),
                       pltpu.VMEM((ECB, DH), jnp.float32),
                       pltpu.VMEM((CET, DH, 128), jnp.float32),
                       pltpu.SemaphoreType.DMA,
                       pltpu.SemaphoreType.DMA],
        compiler_params=pltpu.CompilerParams(use_tc_tiling_on_sc=False,
                                             needs_layout_passes=False),
    )
    def k(xs_hbm, send_hbm, out_hbm, sidx_v, xg_v, xgt_v, isem, gsem):
        wid = lax.axis_index("s") * NC + lax.axis_index("c")

        for j in range(CPT):
            c = wid + NW * j
            off = jnp.where(c < TOTCH, c * ECB, 0)
            pltpu.async_copy(send_hbm.at[pl.ds(off, ECB)],
                             sidx_v.at[pl.ds(j * ECB, ECB)], isem)
        pltpu.make_async_copy(send_hbm.at[pl.ds(0, CPT * ECB)],
                              sidx_v, isem).wait()

        iota = lax.iota(jnp.int32, 16)

        def do_chunk(j, carry):
            c = wid + NW * j

            @pl.when(c < TOTCH)
            def _():
                pltpu.async_copy(
                    xs_hbm.at[sidx_v.at[pl.ds(j * ECB, ECB)]], xg_v,
                    gsem).wait()

                def group(g, carry2):
                    eidx = g * 16 + iota
                    ktile = g // 8
                    coff = (g % 8) * 16
                    for f in range(DH):
                        v = plsc.load_gather(
                            xg_v, [eidx, jnp.full((16,), f, jnp.int32)])
                        xgt_v[ktile, f, pl.ds(coff, 16)] = v
                    return carry2

                lax.fori_loop(0, ECB // 16, group, 0)
                pltpu.sync_copy(xgt_v, out_hbm.at[pl.ds(CET * c, CET)])

            return carry

        lax.fori_loop(0, CPT, do_chunk, 0)

    return k(xs, send)


def _scatter_sc(m3t, rec):
    mesh = plsc.VectorSubcoreMesh(core_axis_name="c", subcore_axis_name="s")

    @functools.partial(
        pl.kernel,
        out_type=jax.ShapeDtypeStruct((NC * N, DH), jnp.float32),
        mesh=mesh,
        scratch_types=[pltpu.VMEM_SHARED((N, DH), jnp.float32),
                       pltpu.VMEM((RPT, DH), jnp.float32),
                       pltpu.VMEM((CPT, ECB), jnp.int32),
                       pltpu.VMEM((CET, DH, 128), jnp.float32),
                       pltpu.VMEM((ECB, DH), jnp.float32),
                       pltpu.SemaphoreType.DMA,
                       pltpu.SemaphoreType.DMA],
        compiler_params=pltpu.CompilerParams(use_tc_tiling_on_sc=False,
                                             needs_layout_passes=False),
    )
    def k(m3_hbm, rec_hbm, out_hbm, agg_sh, zrows_v, recb_v, m3t_v, m3_v,
          rsem, lsem):
        cid = lax.axis_index("c")
        sid = lax.axis_index("s")
        wid = sid * NC + cid

        for j in range(CPT):
            c = wid + NW * j
            off = jnp.where(c < TOTCH, c * ECB, 0)
            pltpu.async_copy(rec_hbm.at[pl.ds(off, ECB)], recb_v.at[j], rsem)
        for j in range(CPT):
            pltpu.make_async_copy(rec_hbm.at[pl.ds(0, ECB)],
                                  recb_v.at[j], rsem).wait()

        def zero_body(r, carry):
            zrows_v[r, :] = jnp.zeros((DH,), jnp.float32)
            return carry

        lax.fori_loop(0, RPT, zero_body, 0)
        pltpu.sync_copy(zrows_v, agg_sh.at[pl.ds(sid * RPT, RPT)])
        plsc.subcore_barrier()

        iota = lax.iota(jnp.int32, 16)

        def do_chunk(j, carry):
            c = wid + NW * j

            @pl.when(c < TOTCH)
            def _():
                pltpu.async_copy(m3_hbm.at[pl.ds(CET * c, CET)], m3t_v,
                                 lsem).wait()

                def group(g, carry2):
                    eidx = g * 16 + iota
                    ktile = g // 8
                    coff = (g % 8) * 16
                    for f in range(DH):
                        v = m3t_v[ktile, f, pl.ds(coff, 16)]
                        plsc.store_scatter(
                            m3_v, [eidx, jnp.full((16,), f, jnp.int32)], v)
                    return carry2

                lax.fori_loop(0, ECB // 16, group, 0)
                pltpu.sync_copy(m3_v, agg_sh.at[recb_v.at[j]], add=True)

            return carry

        lax.fori_loop(0, CPT, do_chunk, 0)
        plsc.subcore_barrier()
        pltpu.sync_copy(agg_sh.at[pl.ds(sid * RPT, RPT)],
                        out_hbm.at[pl.ds(cid * N + sid * RPT, RPT)])

    return k(m3t, rec)


def kernel(x, edge_index, edge_attr, u, batch, mw1, mb1, mw2, mb2, mw3, mb3,
           nw1, nb1, nw2, nb2, nw3, nb3):
    send = edge_index[0]
    rec = edge_index[1]
    mw1a = mw1[:, :DF]
    mw1b = mw1[:, DF:]
    nw1a = nw1[:, :DF]
    nw1b = nw1[:, DF:]

    eye20 = jnp.eye(CET, dtype=jnp.float32)
    w2k = jnp.kron(eye20, mw2)
    w3k = jnp.kron(eye20, mw3)
    b2k = jnp.tile(mb2, CET).reshape(KB, 1)
    b3k = jnp.tile(mb3, CET).reshape(KB, 1)

    xs = _node_pre(x, mw1a)
    eat = _eat_tc(edge_attr.T, mw1b, mb1.reshape(DH, 1))
    xgt = _gather_sc(xs, send)
    m3t = _edge_tc(eat, xgt, w2k, b2k, w3k, b3k)
    parts = _scatter_sc(m3t, rec)
    h = _node_mlp(x, parts, nw1a, nw1b, nb1.reshape(1, DH),
                  nw2, nb2.reshape(1, DH), nw3, nb3.reshape(1, DH))
    return h
